# Initial kernel scaffold; baseline (speedup 1.0000x reference)
import jax
import jax.numpy as jnp
from jax.experimental import pallas as pl


def kernel(x_task, x_vm, ei_dep, ei_rev_dep, ei_can, ei_rev_can, ea_dep, ea_rev_dep, ea_can, ea_rev_can, Wt, bt, Wv, bv, Wsrc, Wdst, Asrc, Adst, Bias, W1, b1, W2, b2):
    raise NotImplementedError("write your pallas kernel here")


# speedup vs baseline: 22.7514x; 22.7514x over previous
import functools

import jax
import jax.numpy as jnp
from jax import lax
from jax.experimental import pallas as pl
from jax.experimental.pallas import tpu as pltpu
from jax.experimental.pallas import tpu_sc as plsc

F32 = jnp.float32
I32 = jnp.int32

HID = 64
NT, NV = 50000, 1000
BN = 256
NTP = 50176
NVP = 1024
NBT = NTP // BN
RT, RV = 51200, 2048
K = 128
NW = 32
ZR = 128
EP_TT = 802816
EP_TV = 401408
SW = 48
MW = 32
DW = 16


@functools.lru_cache(None)
def _edge_pass():
    rpt = RT // 16
    nz = rpt // ZR
    mesh = plsc.VectorSubcoreMesh(core_axis_name="c", subcore_axis_name="s",
                                  num_cores=2, num_subcores=16)

    def body(st0, dt0, st1, dt1, st2, dt2, st3, dt3, ecat, ni_arr,
             out_msg,
             nbuf, idx_s, idx_dg, idx_d, srows, drows, msg, zbuf,
             acc, sem_s, sem_d):
        c = lax.axis_index("c")
        s = lax.axis_index("s")
        pltpu.sync_copy(ni_arr, nbuf)
        niv = nbuf[...]
        zero16 = jnp.zeros((16,), F32)
        for i in range(ZR):
            zbuf[i, 0:16] = zero16
            zbuf[i, 16:32] = zero16
        lane = lax.iota(I32, 16)

        types = [(st0, dt0), (st1, dt1), (st2, dt2), (st3, dt3)]
        for t, (st, dt) in enumerate(types):
            ni = niv[t]

            def zloop(i, carry):
                pltpu.sync_copy(zbuf, acc.at[pl.ds(s * rpt + i * ZR, ZR)])
                return carry
            lax.fori_loop(0, nz, zloop, 0)
            plsc.subcore_barrier()

            def eloop(it, carry):
                base = (s * (2 * ni) + it) * K
                pltpu.sync_copy(ecat.at[4 * t + c, pl.ds(base, K)], idx_s)
                pltpu.sync_copy(ecat.at[4 * t + 2 + c, pl.ds(base, K)], idx_dg)
                coff = c * NTP
                for g2 in range(K // 16):
                    sl = pl.ds(g2 * 16, 16)
                    idx_d[sl] = idx_dg[sl] - coff
                cp1 = pltpu.async_copy(st.at[idx_s], srows, sem_s)
                cp2 = pltpu.async_copy(dt.at[idx_dg], drows, sem_d)
                cp1.wait()
                cp2.wait()
                for g in range(K // 16):
                    r_idx = lane + (g * 16)
                    dst16 = idx_d[pl.ds(g * 16, 16)]
                    for h in range(2):
                        a_s = plsc.load_gather(srows, [r_idx, jnp.full((16,), 32 + h, I32)])
                        a_d = plsc.load_gather(drows, [r_idx, jnp.full((16,), h, I32)])
                        e = a_s + a_d
                        e = jnp.where(e >= 0.0, e, e * F32(0.2))
                        w = jnp.exp(e)
                        for ch in range(16):
                            col = jnp.full((16,), 16 * h + ch, I32)
                            hv = plsc.load_gather(srows, [r_idx, col])
                            plsc.store_scatter(msg, [r_idx, col], hv * w)
                pltpu.sync_copy(msg, acc.at[idx_d], add=True)
                return carry
            lax.fori_loop(0, 2 * ni, eloop, 0)
            plsc.subcore_barrier()
            pltpu.sync_copy(acc.at[pl.ds(s * rpt, rpt)],
                            out_msg.at[t, c, pl.ds(s * rpt, rpt)])

    return pl.kernel(
        body,
        out_type=jax.ShapeDtypeStruct((4, 2, RT, MW), F32),
        mesh=mesh,
        compiler_params=pltpu.CompilerParams(needs_layout_passes=False,
                                             use_tc_tiling_on_sc=False),
        scratch_types=[
            pltpu.VMEM((16,), I32),
            pltpu.VMEM((K,), I32),
            pltpu.VMEM((K,), I32),
            pltpu.VMEM((K,), I32),
            pltpu.VMEM((K, SW), F32),
            pltpu.VMEM((K, DW), F32),
            pltpu.VMEM((K, MW), F32),
            pltpu.VMEM((ZR, MW), F32),
            pltpu.VMEM_SHARED((RT, MW), F32),
            pltpu.SemaphoreType.DMA,
            pltpu.SemaphoreType.DMA,
        ],
    )


@functools.lru_cache(None)
def _den_pass():
    rpt = RT // 16
    nz = rpt // ZR
    mesh = plsc.VectorSubcoreMesh(core_axis_name="c", subcore_axis_name="s",
                                  num_cores=2, num_subcores=16)

    def body(st0, dt0, st1, dt1, st2, dt2, st3, dt3, ecat, ni_arr,
             out_den,
             nbuf, idx_s, idx_dg, idx_d, srows, drows, dmsg, zbuf,
             den, sem_s, sem_d):
        c = lax.axis_index("c")
        s = lax.axis_index("s")
        pltpu.sync_copy(ni_arr, nbuf)
        niv = nbuf[...]
        zero16 = jnp.zeros((16,), F32)
        for i in range(ZR):
            zbuf[i, 0:16] = zero16
        for i in range(K):
            dmsg[i, 0:16] = zero16
        lane = lax.iota(I32, 16)

        types = [(st0, dt0), (st1, dt1), (st2, dt2), (st3, dt3)]
        for t, (st, dt) in enumerate(types):
            ni = niv[t]

            def zloop(i, carry):
                pltpu.sync_copy(zbuf, den.at[pl.ds(s * rpt + i * ZR, ZR)])
                return carry
            lax.fori_loop(0, nz, zloop, 0)
            plsc.subcore_barrier()

            def eloop(it, carry):
                base = (s * (2 * ni) + it) * K
                pltpu.sync_copy(ecat.at[4 * t + c, pl.ds(base, K)], idx_s)
                pltpu.sync_copy(ecat.at[4 * t + 2 + c, pl.ds(base, K)], idx_dg)
                coff = c * NTP
                for g2 in range(K // 16):
                    sl = pl.ds(g2 * 16, 16)
                    idx_d[sl] = idx_dg[sl] - coff
                cp1 = pltpu.async_copy(st.at[idx_s], srows, sem_s)
                cp2 = pltpu.async_copy(dt.at[idx_dg], drows, sem_d)
                cp1.wait()
                cp2.wait()
                for g in range(K // 16):
                    r_idx = lane + (g * 16)
                    for h in range(2):
                        a_s = plsc.load_gather(srows, [r_idx, jnp.full((16,), 32 + h, I32)])
                        a_d = plsc.load_gather(drows, [r_idx, jnp.full((16,), h, I32)])
                        e = a_s + a_d
                        e = jnp.where(e >= 0.0, e, e * F32(0.2))
                        w = jnp.exp(e)
                        plsc.store_scatter(dmsg, [r_idx, jnp.full((16,), h, I32)], w)
                pltpu.sync_copy(dmsg, den.at[idx_d], add=True)
                return carry
            lax.fori_loop(0, 2 * ni, eloop, 0)
            plsc.subcore_barrier()
            pltpu.sync_copy(den.at[pl.ds(s * rpt, rpt)],
                            out_den.at[t, c, pl.ds(s * rpt, rpt)])

    return pl.kernel(
        body,
        out_type=jax.ShapeDtypeStruct((4, 2, RT, DW), F32),
        mesh=mesh,
        compiler_params=pltpu.CompilerParams(needs_layout_passes=False,
                                             use_tc_tiling_on_sc=False),
        scratch_types=[
            pltpu.VMEM((16,), I32),
            pltpu.VMEM((K,), I32),
            pltpu.VMEM((K,), I32),
            pltpu.VMEM((K,), I32),
            pltpu.VMEM((K, SW), F32),
            pltpu.VMEM((K, DW), F32),
            pltpu.VMEM((K, DW), F32),
            pltpu.VMEM((ZR, DW), F32),
            pltpu.VMEM_SHARED((RT, DW), F32),
            pltpu.SemaphoreType.DMA,
            pltpu.SemaphoreType.DMA,
        ],
    )


def _dot(a, b):
    return lax.dot_general(a, b, (((1,), (0,)), ((), ())),
                           preferred_element_type=F32)


def _encoder_body(x_ref, w_ref, b_ref, o_ref):
    o_ref[...] = jax.nn.relu(_dot(x_ref[...], w_ref[...]) + b_ref[...])


def _encode(x, w, b, nblocks, bn):
    din = x.shape[1]
    return pl.pallas_call(
        _encoder_body,
        grid=(nblocks,),
        in_specs=[
            pl.BlockSpec((bn, din), lambda i: (i, 0)),
            pl.BlockSpec((din, HID), lambda i: (0, 0)),
            pl.BlockSpec((1, HID), lambda i: (0, 0)),
        ],
        out_specs=pl.BlockSpec((bn, HID), lambda i: (i, 0)),
        out_shape=jax.ShapeDtypeStruct((nblocks * bn, HID), F32),
    )(x, w, b)


def _avals(h, a_ref, bn):
    return jnp.sum((h * a_ref[...]).reshape(bn, 4, 16), axis=2)


def _pack_src(st_ref, h, asv, bn):
    for c in range(2):
        st_ref[c, :, 0:32] = h[:, 32 * c:32 * c + 32]
        st_ref[c, :, 32:34] = asv[:, 2 * c:2 * c + 2]
        st_ref[c, :, 34:48] = jnp.zeros((bn, 14), F32)


def _pack_dst(dt_ref, adv, bn):
    for c in range(2):
        dt_ref[c, :, 0:2] = adv[:, 2 * c:2 * c + 2]
        dt_ref[c, :, 2:16] = jnp.zeros((bn, 14), F32)


def _proj_task_body(x_ref, w0, w1, w2, w3, a0s, a0d, a1s, a1d, a2s, a3d,
                    st0, dt0, st1, dt1, st2, dt3):
    x = x_ref[...]
    h0 = _dot(x, w0[...])
    _pack_src(st0, h0, _avals(h0, a0s, BN), BN)
    _pack_dst(dt0, _avals(h0, a0d, BN), BN)
    h1 = _dot(x, w1[...])
    _pack_src(st1, h1, _avals(h1, a1s, BN), BN)
    _pack_dst(dt1, _avals(h1, a1d, BN), BN)
    h2 = _dot(x, w2[...])
    _pack_src(st2, h2, _avals(h2, a2s, BN), BN)
    h3 = _dot(x, w3[...])
    _pack_dst(dt3, _avals(h3, a3d, BN), BN)


def _proj_task(xt, w0, w1, w2, w3, a0s, a0d, a1s, a1d, a2s, a3d):
    wspec = pl.BlockSpec((HID, HID), lambda i: (0, 0))
    aspec = pl.BlockSpec((1, HID), lambda i: (0, 0))
    st_spec = pl.BlockSpec((2, BN, SW), lambda i: (0, i, 0))
    dt_spec = pl.BlockSpec((2, BN, DW), lambda i: (0, i, 0))
    st_shape = jax.ShapeDtypeStruct((2, NTP, SW), F32)
    dt_shape = jax.ShapeDtypeStruct((2, NTP, DW), F32)
    return pl.pallas_call(
        _proj_task_body,
        grid=(NBT,),
        in_specs=[pl.BlockSpec((BN, HID), lambda i: (i, 0))]
                 + [wspec] * 4 + [aspec] * 6,
        out_specs=[st_spec, dt_spec, st_spec, dt_spec, st_spec, dt_spec],
        out_shape=[st_shape, dt_shape, st_shape, dt_shape, st_shape, dt_shape],
    )(xt, w0, w1, w2, w3, a0s, a0d, a1s, a1d, a2s, a3d)


def _proj_vm_body(x_ref, w2d, w3s, a2d, a3s, dt2, st3):
    x = x_ref[...]
    h2 = _dot(x, w2d[...])
    _pack_dst(dt2, _avals(h2, a2d, NVP), NVP)
    h3 = _dot(x, w3s[...])
    _pack_src(st3, h3, _avals(h3, a3s, NVP), NVP)


def _proj_vm(xv, w2d, w3s, a2d, a3s):
    return pl.pallas_call(
        _proj_vm_body,
        grid=(1,),
        in_specs=[pl.BlockSpec((NVP, HID), lambda i: (0, 0)),
                  pl.BlockSpec((HID, HID), lambda i: (0, 0)),
                  pl.BlockSpec((HID, HID), lambda i: (0, 0)),
                  pl.BlockSpec((1, HID), lambda i: (0, 0)),
                  pl.BlockSpec((1, HID), lambda i: (0, 0))],
        out_specs=[pl.BlockSpec((2, NVP, DW), lambda i: (0, 0, 0)),
                   pl.BlockSpec((2, NVP, SW), lambda i: (0, 0, 0))],
        out_shape=[jax.ShapeDtypeStruct((2, NVP, DW), F32),
                   jax.ShapeDtypeStruct((2, NVP, SW), F32)],
    )(xv, w2d, w3s, a2d, a3s)


def _halves_div(acc_ref, den_ref, bn, st_ref=None, dt_ref=None):
    outs = []
    for c in range(2):
        num = acc_ref[c, :, 0:32]
        den = den_ref[c, :, 0:2]
        if st_ref is not None:
            hh = st_ref[c, :, 0:32]
            asv = st_ref[c, :, 32:34]
            adv = dt_ref[c, :, 0:2]
            e = asv + adv
            w = jnp.exp(jnp.where(e >= 0.0, e, e * F32(0.2)))
            num = num + (hh.reshape(bn, 2, 16) * w[:, :, None]).reshape(bn, 32)
            den = den + w
        o = num.reshape(bn, 2, 16) / (den[:, :, None] + F32(1e-16))
        outs.append(o.reshape(bn, 32))
    return jnp.concatenate(outs, axis=1)


def _comb_task_body(a0, d0, a1, d1, a3, d3, st0, dt0, st1, dt1,
                    x_ref, b0, b1, b3, o_ref):
    i = pl.program_id(0)
    o0 = _halves_div(a0, d0, BN, st0, dt0)
    o1 = _halves_div(a1, d1, BN, st1, dt1)
    o3 = _halves_div(a3, d3, BN)
    t = jax.nn.relu(o0 + b0[...] + o1 + b1[...] + o3 + b3[...])
    row = i * BN + lax.broadcasted_iota(I32, (BN, 1), 0)
    o_ref[...] = jnp.where(row < NT, x_ref[...] + t, 0.0)


def _comb_task(a0, d0, a1, d1, a3, d3, st0, dt0, st1, dt1, xt, b0, b1, b3):
    acc_spec = pl.BlockSpec((2, BN, MW), lambda i: (0, i, 0))
    den_spec = pl.BlockSpec((2, BN, DW), lambda i: (0, i, 0))
    st_spec = pl.BlockSpec((2, BN, SW), lambda i: (0, i, 0))
    dt_spec = pl.BlockSpec((2, BN, DW), lambda i: (0, i, 0))
    bspec = pl.BlockSpec((1, HID), lambda i: (0, 0))
    return pl.pallas_call(
        _comb_task_body,
        grid=(NBT,),
        in_specs=[acc_spec, den_spec, acc_spec, den_spec, acc_spec, den_spec,
                  st_spec, dt_spec, st_spec, dt_spec,
                  pl.BlockSpec((BN, HID), lambda i: (i, 0)),
                  bspec, bspec, bspec],
        out_specs=pl.BlockSpec((BN, HID), lambda i: (i, 0)),
        out_shape=jax.ShapeDtypeStruct((NTP, HID), F32),
    )(a0, d0, a1, d1, a3, d3, st0, dt0, st1, dt1, xt, b0, b1, b3)


def _comb_vm_body(a2, d2, x_ref, b2, o_ref):
    o2 = _halves_div(a2, d2, NVP)
    t = jax.nn.relu(o2 + b2[...])
    row = lax.broadcasted_iota(I32, (NVP, 1), 0)
    o_ref[...] = jnp.where(row < NV, x_ref[...] + t, 0.0)


def _comb_vm(a2, d2, xv, b2):
    return pl.pallas_call(
        _comb_vm_body,
        grid=(1,),
        in_specs=[pl.BlockSpec((2, NVP, MW), lambda i: (0, 0, 0)),
                  pl.BlockSpec((2, NVP, DW), lambda i: (0, 0, 0)),
                  pl.BlockSpec((NVP, HID), lambda i: (0, 0)),
                  pl.BlockSpec((1, HID), lambda i: (0, 0))],
        out_specs=pl.BlockSpec((NVP, HID), lambda i: (0, 0)),
        out_shape=jax.ShapeDtypeStruct((NVP, HID), F32),
    )(a2, d2, xv, b2)


def _pool_mlp_body(x_ref, w1, b1, w2, b2, o_ref, acc_ref):
    i = pl.program_id(0)

    @pl.when(i == 0)
    def _():
        acc_ref[...] = jnp.zeros_like(acc_ref)

    acc_ref[...] += jnp.sum(x_ref[...], axis=0, keepdims=True)

    @pl.when(i == NBT - 1)
    def _():
        m = acc_ref[...] * F32(1.0 / NT)
        h = jax.nn.relu(_dot(m, w1[...]) + b1[...])
        o_ref[...] = _dot(h, w2[...]) + b2[...]


def _pool_mlp(xt, w1, b1, w2, b2):
    return pl.pallas_call(
        _pool_mlp_body,
        grid=(NBT,),
        in_specs=[pl.BlockSpec((BN, HID), lambda i: (i, 0)),
                  pl.BlockSpec((HID, 32), lambda i: (0, 0)),
                  pl.BlockSpec((1, 32), lambda i: (0, 0)),
                  pl.BlockSpec((32, 32), lambda i: (0, 0)),
                  pl.BlockSpec((1, 32), lambda i: (0, 0))],
        out_specs=pl.BlockSpec((1, 32), lambda i: (0, 0)),
        out_shape=jax.ShapeDtypeStruct((1, 32), F32),
        scratch_shapes=[pltpu.VMEM((1, HID), F32)],
    )(xt, w1, b1, w2, b2)


def _prep_edges(ei, nsrc_dummy, ndst_dummy):
    e = ei.shape[1]
    src = jnp.concatenate([ei[0], jnp.full((EP_TT - e,), nsrc_dummy, I32)])
    dst = jnp.concatenate([ei[1], jnp.full((EP_TT - e,), ndst_dummy, I32)])
    return jnp.stack([src, src + NTP, dst, dst + NTP])


def _padvm(t, w):
    return jnp.pad(t, ((0, 0), (0, NTP - NVP), (0, 0))).reshape(2 * NTP, w)


def kernel(x_task, x_vm, ei_dep, ei_rev_dep, ei_can, ei_rev_can,
           ea_dep, ea_rev_dep, ea_can, ea_rev_can,
           Wt, bt, Wv, bv, Wsrc, Wdst, Asrc, Adst, Bias, W1, b1, W2, b2):
    xt0 = _encode(jnp.pad(x_task, ((0, NTP - NT), (0, 0))), Wt,
                  bt.reshape(1, HID), NBT, BN)
    xv0 = _encode(jnp.pad(x_vm, ((0, NVP - NV), (0, 0))), Wv,
                  bv.reshape(1, HID), 1, NVP)

    ecat = jnp.concatenate([
        _prep_edges(ei_dep, NT, NT),
        _prep_edges(ei_rev_dep, NT, NT),
        _prep_edges(ei_can, NT, NV),
        _prep_edges(ei_rev_can, NV, NT)], axis=0)
    ni_base = jnp.array([EP_TT // (NW * K), EP_TT // (NW * K),
                         EP_TV // (NW * K), EP_TV // (NW * K)] + [0] * 12, I32)
    ni_all = jnp.where(ei_dep[0, :16] < -1, 0, ni_base)

    ek = _edge_pass()
    dk = _den_pass()

    def a2d(a):
        return a.reshape(1, HID)

    def lbody(l, carry):
        xt, xv = carry
        st0, dt0, st1, dt1, st2, dt3 = _proj_task(
            xt, Wsrc[l, 0], Wsrc[l, 1], Wsrc[l, 2], Wdst[l, 3],
            a2d(Asrc[l, 0]), a2d(Adst[l, 0]), a2d(Asrc[l, 1]),
            a2d(Adst[l, 1]), a2d(Asrc[l, 2]), a2d(Adst[l, 3]))
        dt2, st3 = _proj_vm(xv, Wdst[l, 2], Wsrc[l, 3],
                            a2d(Adst[l, 2]), a2d(Asrc[l, 3]))
        sc_args = (
            st0.reshape(2 * NTP, SW), dt0.reshape(2 * NTP, DW),
            st1.reshape(2 * NTP, SW), dt1.reshape(2 * NTP, DW),
            st2.reshape(2 * NTP, SW), _padvm(dt2, DW),
            _padvm(st3, SW), dt3.reshape(2 * NTP, DW),
            ecat, ni_all)
        out_msg = ek(*sc_args)
        out_den = dk(*sc_args)
        xt_new = _comb_task(out_msg[0, :, :NTP], out_den[0][:, :NTP],
                            out_msg[1, :, :NTP], out_den[1][:, :NTP],
                            out_msg[3, :, :NTP], out_den[3][:, :NTP],
                            st0, dt0, st1, dt1, xt,
                            a2d(Bias[l, 0]), a2d(Bias[l, 1]),
                            a2d(Bias[l, 3]))
        xv_new = _comb_vm(out_msg[2, :, :NVP], out_den[2][:, :NVP],
                          xv, a2d(Bias[l, 2]))
        return xt_new, xv_new

    nlayers = jnp.where(ei_dep[0, 0] < -1, 3, 4)
    xt, xv = lax.fori_loop(0, nlayers, lbody, (xt0, xv0))
    return _pool_mlp(xt, W1, b1.reshape(1, 32), W2, b2.reshape(1, 32))


# speedup vs baseline: 23.0370x; 1.0126x over previous
import functools

import jax
import jax.numpy as jnp
from jax import lax
from jax.experimental import pallas as pl
from jax.experimental.pallas import tpu as pltpu
from jax.experimental.pallas import tpu_sc as plsc

F32 = jnp.float32
I32 = jnp.int32

HID = 64
NT, NV = 50000, 1000
BN = 256
NTP = 50176
NVP = 1024
NBT = NTP // BN
RT, RV = 51200, 2048
K = 128
NW = 32
ZR = 128
EP_TT = 802816
EP_TV = 401408
SW = 48
MW = 32
DW = 16


@functools.lru_cache(None)
def _edge_pass():
    rpt = RT // 16
    nz = rpt // ZR
    mesh = plsc.VectorSubcoreMesh(core_axis_name="c", subcore_axis_name="s",
                                  num_cores=2, num_subcores=16)

    def body(st0, dt0, st1, dt1, st2, dt2, st3, dt3, ecat, ni_arr,
             out_msg,
             nbuf, idx_s, idx_dg, idx_d, srows, drows, msg, zbuf,
             acc, sem_s, sem_d):
        c = lax.axis_index("c")
        s = lax.axis_index("s")
        pltpu.sync_copy(ni_arr, nbuf)
        niv = nbuf[...]
        zero16 = jnp.zeros((16,), F32)
        for i in range(ZR):
            zbuf[i, 0:16] = zero16
            zbuf[i, 16:32] = zero16
        lane = lax.iota(I32, 16)

        types = [(st0, dt0), (st1, dt1), (st2, dt2), (st3, dt3)]
        for t, (st, dt) in enumerate(types):
            ni = niv[t]

            def zloop(i, carry):
                pltpu.sync_copy(zbuf, acc.at[pl.ds(s * rpt + i * ZR, ZR)])
                return carry
            lax.fori_loop(0, nz, zloop, 0)
            plsc.subcore_barrier()

            def eloop(it, carry):
                base = (s * (2 * ni) + it) * K
                pltpu.sync_copy(ecat.at[4 * t + c, pl.ds(base, K)], idx_s)
                pltpu.sync_copy(ecat.at[4 * t + 2 + c, pl.ds(base, K)], idx_dg)
                coff = c * NTP
                for g2 in range(K // 16):
                    sl = pl.ds(g2 * 16, 16)
                    idx_d[sl] = idx_dg[sl] - coff
                cp1 = pltpu.async_copy(st.at[idx_s], srows, sem_s)
                cp2 = pltpu.async_copy(dt.at[idx_dg], drows, sem_d)
                cp1.wait()
                cp2.wait()
                for g in range(K // 16):
                    r_idx = lane + (g * 16)
                    dst16 = idx_d[pl.ds(g * 16, 16)]
                    for h in range(2):
                        a_s = plsc.load_gather(srows, [r_idx, jnp.full((16,), 32 + h, I32)])
                        a_d = plsc.load_gather(drows, [r_idx, jnp.full((16,), h, I32)])
                        e = a_s + a_d
                        e = jnp.where(e >= 0.0, e, e * F32(0.2))
                        w = jnp.exp(e)
                        for ch in range(16):
                            col = jnp.full((16,), 16 * h + ch, I32)
                            hv = plsc.load_gather(srows, [r_idx, col])
                            plsc.store_scatter(msg, [r_idx, col], hv * w)
                pltpu.sync_copy(msg, acc.at[idx_d], add=True)
                return carry
            lax.fori_loop(0, 2 * ni, eloop, 0)
            plsc.subcore_barrier()
            pltpu.sync_copy(acc.at[pl.ds(s * rpt, rpt)],
                            out_msg.at[t, c, pl.ds(s * rpt, rpt)])

    return pl.kernel(
        body,
        out_type=jax.ShapeDtypeStruct((4, 2, RT, MW), F32),
        mesh=mesh,
        compiler_params=pltpu.CompilerParams(needs_layout_passes=False,
                                             use_tc_tiling_on_sc=False),
        scratch_types=[
            pltpu.VMEM((16,), I32),
            pltpu.VMEM((K,), I32),
            pltpu.VMEM((K,), I32),
            pltpu.VMEM((K,), I32),
            pltpu.VMEM((K, SW), F32),
            pltpu.VMEM((K, DW), F32),
            pltpu.VMEM((K, MW), F32),
            pltpu.VMEM((ZR, MW), F32),
            pltpu.VMEM_SHARED((RT, MW), F32),
            pltpu.SemaphoreType.DMA,
            pltpu.SemaphoreType.DMA,
        ],
    )


@functools.lru_cache(None)
def _den_pass():
    rpt = RT // 16
    nz = rpt // ZR
    mesh = plsc.VectorSubcoreMesh(core_axis_name="c", subcore_axis_name="s",
                                  num_cores=2, num_subcores=16)

    def body(st0, dt0, st1, dt1, st2, dt2, st3, dt3, ecat, ni_arr,
             out_den,
             nbuf, idx_s, idx_dg, idx_d, srows, drows, dmsg, zbuf,
             den, sem_s, sem_d):
        c = lax.axis_index("c")
        s = lax.axis_index("s")
        pltpu.sync_copy(ni_arr, nbuf)
        niv = nbuf[...]
        zero16 = jnp.zeros((16,), F32)
        for i in range(ZR):
            zbuf[i, 0:16] = zero16
        for i in range(K):
            dmsg[i, 0:16] = zero16
        lane = lax.iota(I32, 16)

        types = [(st0, dt0), (st1, dt1), (st2, dt2), (st3, dt3)]
        for t, (st, dt) in enumerate(types):
            ni = niv[t]

            def zloop(i, carry):
                pltpu.sync_copy(zbuf, den.at[pl.ds(s * rpt + i * ZR, ZR)])
                return carry
            lax.fori_loop(0, nz, zloop, 0)
            plsc.subcore_barrier()

            def eloop(it, carry):
                base = (s * (2 * ni) + it) * K
                pltpu.sync_copy(ecat.at[4 * t + c, pl.ds(base, K)], idx_s)
                pltpu.sync_copy(ecat.at[4 * t + 2 + c, pl.ds(base, K)], idx_dg)
                coff = c * NTP
                for g2 in range(K // 16):
                    sl = pl.ds(g2 * 16, 16)
                    idx_d[sl] = idx_dg[sl] - coff
                cp1 = pltpu.async_copy(st.at[idx_s], srows, sem_s)
                cp2 = pltpu.async_copy(dt.at[idx_dg], drows, sem_d)
                cp1.wait()
                cp2.wait()
                for g in range(K // 16):
                    r_idx = lane + (g * 16)
                    for h in range(2):
                        a_s = plsc.load_gather(srows, [r_idx, jnp.full((16,), h, I32)])
                        a_d = plsc.load_gather(drows, [r_idx, jnp.full((16,), h, I32)])
                        e = a_s + a_d
                        e = jnp.where(e >= 0.0, e, e * F32(0.2))
                        w = jnp.exp(e)
                        plsc.store_scatter(dmsg, [r_idx, jnp.full((16,), h, I32)], w)
                pltpu.sync_copy(dmsg, den.at[idx_d], add=True)
                return carry
            lax.fori_loop(0, 2 * ni, eloop, 0)
            plsc.subcore_barrier()
            pltpu.sync_copy(den.at[pl.ds(s * rpt, rpt)],
                            out_den.at[t, c, pl.ds(s * rpt, rpt)])

    return pl.kernel(
        body,
        out_type=jax.ShapeDtypeStruct((4, 2, RT, DW), F32),
        mesh=mesh,
        compiler_params=pltpu.CompilerParams(needs_layout_passes=False,
                                             use_tc_tiling_on_sc=False),
        scratch_types=[
            pltpu.VMEM((16,), I32),
            pltpu.VMEM((K,), I32),
            pltpu.VMEM((K,), I32),
            pltpu.VMEM((K,), I32),
            pltpu.VMEM((K, DW), F32),
            pltpu.VMEM((K, DW), F32),
            pltpu.VMEM((K, DW), F32),
            pltpu.VMEM((ZR, DW), F32),
            pltpu.VMEM_SHARED((RT, DW), F32),
            pltpu.SemaphoreType.DMA,
            pltpu.SemaphoreType.DMA,
        ],
    )


def _dot(a, b):
    return lax.dot_general(a, b, (((1,), (0,)), ((), ())),
                           preferred_element_type=F32)


def _encoder_body(x_ref, w_ref, b_ref, o_ref):
    o_ref[...] = jax.nn.relu(_dot(x_ref[...], w_ref[...]) + b_ref[...])


def _encode(x, w, b, nblocks, bn):
    din = x.shape[1]
    return pl.pallas_call(
        _encoder_body,
        grid=(nblocks,),
        in_specs=[
            pl.BlockSpec((bn, din), lambda i: (i, 0)),
            pl.BlockSpec((din, HID), lambda i: (0, 0)),
            pl.BlockSpec((1, HID), lambda i: (0, 0)),
        ],
        out_specs=pl.BlockSpec((bn, HID), lambda i: (i, 0)),
        out_shape=jax.ShapeDtypeStruct((nblocks * bn, HID), F32),
    )(x, w, b)


def _avals(h, a_ref, bn):
    return jnp.sum((h * a_ref[...]).reshape(bn, 4, 16), axis=2)


def _pack_src(st_ref, h, asv, bn):
    for c in range(2):
        st_ref[c, :, 0:32] = h[:, 32 * c:32 * c + 32]
        st_ref[c, :, 32:34] = asv[:, 2 * c:2 * c + 2]
        st_ref[c, :, 34:48] = jnp.zeros((bn, 14), F32)


def _pack_dst(dt_ref, adv, bn):
    for c in range(2):
        dt_ref[c, :, 0:2] = adv[:, 2 * c:2 * c + 2]
        dt_ref[c, :, 2:16] = jnp.zeros((bn, 14), F32)


def _proj_task_body(x_ref, w0, w1, w2, w3, a0s, a0d, a1s, a1d, a2s, a3d,
                    st0, dt0, st1, dt1, st2, dt3, sa0, sa1, sa2):
    x = x_ref[...]
    h0 = _dot(x, w0[...])
    as0 = _avals(h0, a0s, BN)
    _pack_src(st0, h0, as0, BN)
    _pack_dst(sa0, as0, BN)
    _pack_dst(dt0, _avals(h0, a0d, BN), BN)
    h1 = _dot(x, w1[...])
    as1 = _avals(h1, a1s, BN)
    _pack_src(st1, h1, as1, BN)
    _pack_dst(sa1, as1, BN)
    _pack_dst(dt1, _avals(h1, a1d, BN), BN)
    h2 = _dot(x, w2[...])
    as2 = _avals(h2, a2s, BN)
    _pack_src(st2, h2, as2, BN)
    _pack_dst(sa2, as2, BN)
    h3 = _dot(x, w3[...])
    _pack_dst(dt3, _avals(h3, a3d, BN), BN)


def _proj_task(xt, w0, w1, w2, w3, a0s, a0d, a1s, a1d, a2s, a3d):
    wspec = pl.BlockSpec((HID, HID), lambda i: (0, 0))
    aspec = pl.BlockSpec((1, HID), lambda i: (0, 0))
    st_spec = pl.BlockSpec((2, BN, SW), lambda i: (0, i, 0))
    dt_spec = pl.BlockSpec((2, BN, DW), lambda i: (0, i, 0))
    st_shape = jax.ShapeDtypeStruct((2, NTP, SW), F32)
    dt_shape = jax.ShapeDtypeStruct((2, NTP, DW), F32)
    return pl.pallas_call(
        _proj_task_body,
        grid=(NBT,),
        in_specs=[pl.BlockSpec((BN, HID), lambda i: (i, 0))]
                 + [wspec] * 4 + [aspec] * 6,
        out_specs=[st_spec, dt_spec, st_spec, dt_spec, st_spec, dt_spec,
                   dt_spec, dt_spec, dt_spec],
        out_shape=[st_shape, dt_shape, st_shape, dt_shape, st_shape, dt_shape,
                   dt_shape, dt_shape, dt_shape],
    )(xt, w0, w1, w2, w3, a0s, a0d, a1s, a1d, a2s, a3d)


def _proj_vm_body(x_ref, w2d, w3s, a2d, a3s, dt2, st3, sa3):
    x = x_ref[...]
    h2 = _dot(x, w2d[...])
    _pack_dst(dt2, _avals(h2, a2d, NVP), NVP)
    h3 = _dot(x, w3s[...])
    as3 = _avals(h3, a3s, NVP)
    _pack_src(st3, h3, as3, NVP)
    _pack_dst(sa3, as3, NVP)


def _proj_vm(xv, w2d, w3s, a2d, a3s):
    return pl.pallas_call(
        _proj_vm_body,
        grid=(1,),
        in_specs=[pl.BlockSpec((NVP, HID), lambda i: (0, 0)),
                  pl.BlockSpec((HID, HID), lambda i: (0, 0)),
                  pl.BlockSpec((HID, HID), lambda i: (0, 0)),
                  pl.BlockSpec((1, HID), lambda i: (0, 0)),
                  pl.BlockSpec((1, HID), lambda i: (0, 0))],
        out_specs=[pl.BlockSpec((2, NVP, DW), lambda i: (0, 0, 0)),
                   pl.BlockSpec((2, NVP, SW), lambda i: (0, 0, 0)),
                   pl.BlockSpec((2, NVP, DW), lambda i: (0, 0, 0))],
        out_shape=[jax.ShapeDtypeStruct((2, NVP, DW), F32),
                   jax.ShapeDtypeStruct((2, NVP, SW), F32),
                   jax.ShapeDtypeStruct((2, NVP, DW), F32)],
    )(xv, w2d, w3s, a2d, a3s)


def _halves_div(acc_ref, den_ref, bn, st_ref=None, dt_ref=None):
    outs = []
    for c in range(2):
        num = acc_ref[c, :, 0:32]
        den = den_ref[c, :, 0:2]
        if st_ref is not None:
            hh = st_ref[c, :, 0:32]
            asv = st_ref[c, :, 32:34]
            adv = dt_ref[c, :, 0:2]
            e = asv + adv
            w = jnp.exp(jnp.where(e >= 0.0, e, e * F32(0.2)))
            num = num + (hh.reshape(bn, 2, 16) * w[:, :, None]).reshape(bn, 32)
            den = den + w
        o = num.reshape(bn, 2, 16) / (den[:, :, None] + F32(1e-16))
        outs.append(o.reshape(bn, 32))
    return jnp.concatenate(outs, axis=1)


def _comb_task_body(a0, d0, a1, d1, a3, d3, st0, dt0, st1, dt1,
                    x_ref, b0, b1, b3, o_ref):
    i = pl.program_id(0)
    o0 = _halves_div(a0, d0, BN, st0, dt0)
    o1 = _halves_div(a1, d1, BN, st1, dt1)
    o3 = _halves_div(a3, d3, BN)
    t = jax.nn.relu(o0 + b0[...] + o1 + b1[...] + o3 + b3[...])
    row = i * BN + lax.broadcasted_iota(I32, (BN, 1), 0)
    o_ref[...] = jnp.where(row < NT, x_ref[...] + t, 0.0)


def _comb_task(a0, d0, a1, d1, a3, d3, st0, dt0, st1, dt1, xt, b0, b1, b3):
    acc_spec = pl.BlockSpec((2, BN, MW), lambda i: (0, i, 0))
    den_spec = pl.BlockSpec((2, BN, DW), lambda i: (0, i, 0))
    st_spec = pl.BlockSpec((2, BN, SW), lambda i: (0, i, 0))
    dt_spec = pl.BlockSpec((2, BN, DW), lambda i: (0, i, 0))
    bspec = pl.BlockSpec((1, HID), lambda i: (0, 0))
    return pl.pallas_call(
        _comb_task_body,
        grid=(NBT,),
        in_specs=[acc_spec, den_spec, acc_spec, den_spec, acc_spec, den_spec,
                  st_spec, dt_spec, st_spec, dt_spec,
                  pl.BlockSpec((BN, HID), lambda i: (i, 0)),
                  bspec, bspec, bspec],
        out_specs=pl.BlockSpec((BN, HID), lambda i: (i, 0)),
        out_shape=jax.ShapeDtypeStruct((NTP, HID), F32),
    )(a0, d0, a1, d1, a3, d3, st0, dt0, st1, dt1, xt, b0, b1, b3)


def _comb_vm_body(a2, d2, x_ref, b2, o_ref):
    o2 = _halves_div(a2, d2, NVP)
    t = jax.nn.relu(o2 + b2[...])
    row = lax.broadcasted_iota(I32, (NVP, 1), 0)
    o_ref[...] = jnp.where(row < NV, x_ref[...] + t, 0.0)


def _comb_vm(a2, d2, xv, b2):
    return pl.pallas_call(
        _comb_vm_body,
        grid=(1,),
        in_specs=[pl.BlockSpec((2, NVP, MW), lambda i: (0, 0, 0)),
                  pl.BlockSpec((2, NVP, DW), lambda i: (0, 0, 0)),
                  pl.BlockSpec((NVP, HID), lambda i: (0, 0)),
                  pl.BlockSpec((1, HID), lambda i: (0, 0))],
        out_specs=pl.BlockSpec((NVP, HID), lambda i: (0, 0)),
        out_shape=jax.ShapeDtypeStruct((NVP, HID), F32),
    )(a2, d2, xv, b2)


def _pool_mlp_body(x_ref, w1, b1, w2, b2, o_ref, acc_ref):
    i = pl.program_id(0)

    @pl.when(i == 0)
    def _():
        acc_ref[...] = jnp.zeros_like(acc_ref)

    acc_ref[...] += jnp.sum(x_ref[...], axis=0, keepdims=True)

    @pl.when(i == NBT - 1)
    def _():
        m = acc_ref[...] * F32(1.0 / NT)
        h = jax.nn.relu(_dot(m, w1[...]) + b1[...])
        o_ref[...] = _dot(h, w2[...]) + b2[...]


def _pool_mlp(xt, w1, b1, w2, b2):
    return pl.pallas_call(
        _pool_mlp_body,
        grid=(NBT,),
        in_specs=[pl.BlockSpec((BN, HID), lambda i: (i, 0)),
                  pl.BlockSpec((HID, 32), lambda i: (0, 0)),
                  pl.BlockSpec((1, 32), lambda i: (0, 0)),
                  pl.BlockSpec((32, 32), lambda i: (0, 0)),
                  pl.BlockSpec((1, 32), lambda i: (0, 0))],
        out_specs=pl.BlockSpec((1, 32), lambda i: (0, 0)),
        out_shape=jax.ShapeDtypeStruct((1, 32), F32),
        scratch_shapes=[pltpu.VMEM((1, HID), F32)],
    )(xt, w1, b1, w2, b2)


def _prep_edges(ei, nsrc_dummy, ndst_dummy):
    e = ei.shape[1]
    src = jnp.concatenate([ei[0], jnp.full((EP_TT - e,), nsrc_dummy, I32)])
    dst = jnp.concatenate([ei[1], jnp.full((EP_TT - e,), ndst_dummy, I32)])
    return jnp.stack([src, src + NTP, dst, dst + NTP])


def _padvm(t, w):
    return jnp.pad(t, ((0, 0), (0, NTP - NVP), (0, 0))).reshape(2 * NTP, w)


def kernel(x_task, x_vm, ei_dep, ei_rev_dep, ei_can, ei_rev_can,
           ea_dep, ea_rev_dep, ea_can, ea_rev_can,
           Wt, bt, Wv, bv, Wsrc, Wdst, Asrc, Adst, Bias, W1, b1, W2, b2):
    xt0 = _encode(jnp.pad(x_task, ((0, NTP - NT), (0, 0))), Wt,
                  bt.reshape(1, HID), NBT, BN)
    xv0 = _encode(jnp.pad(x_vm, ((0, NVP - NV), (0, 0))), Wv,
                  bv.reshape(1, HID), 1, NVP)

    ecat = jnp.concatenate([
        _prep_edges(ei_dep, NT, NT),
        _prep_edges(ei_rev_dep, NT, NT),
        _prep_edges(ei_can, NT, NV),
        _prep_edges(ei_rev_can, NV, NT)], axis=0)
    ni_base = jnp.array([EP_TT // (NW * K), EP_TT // (NW * K),
                         EP_TV // (NW * K), EP_TV // (NW * K)] + [0] * 12, I32)
    ni_all = jnp.where(ei_dep[0, :16] < -1, 0, ni_base)

    ek = _edge_pass()
    dk = _den_pass()

    def a2d(a):
        return a.reshape(1, HID)

    def lbody(l, carry):
        xt, xv = carry
        st0, dt0, st1, dt1, st2, dt3, sa0, sa1, sa2 = _proj_task(
            xt, Wsrc[l, 0], Wsrc[l, 1], Wsrc[l, 2], Wdst[l, 3],
            a2d(Asrc[l, 0]), a2d(Adst[l, 0]), a2d(Asrc[l, 1]),
            a2d(Adst[l, 1]), a2d(Asrc[l, 2]), a2d(Adst[l, 3]))
        dt2, st3, sa3 = _proj_vm(xv, Wdst[l, 2], Wsrc[l, 3],
                                 a2d(Adst[l, 2]), a2d(Asrc[l, 3]))
        dt0f = dt0.reshape(2 * NTP, DW)
        dt1f = dt1.reshape(2 * NTP, DW)
        dt2f = _padvm(dt2, DW)
        dt3f = dt3.reshape(2 * NTP, DW)
        out_msg = ek(
            st0.reshape(2 * NTP, SW), dt0f,
            st1.reshape(2 * NTP, SW), dt1f,
            st2.reshape(2 * NTP, SW), dt2f,
            _padvm(st3, SW), dt3f,
            ecat, ni_all)
        out_den = dk(
            sa0.reshape(2 * NTP, DW), dt0f,
            sa1.reshape(2 * NTP, DW), dt1f,
            sa2.reshape(2 * NTP, DW), dt2f,
            _padvm(sa3, DW), dt3f,
            ecat, ni_all)
        xt_new = _comb_task(out_msg[0, :, :NTP], out_den[0][:, :NTP],
                            out_msg[1, :, :NTP], out_den[1][:, :NTP],
                            out_msg[3, :, :NTP], out_den[3][:, :NTP],
                            st0, dt0, st1, dt1, xt,
                            a2d(Bias[l, 0]), a2d(Bias[l, 1]),
                            a2d(Bias[l, 3]))
        xv_new = _comb_vm(out_msg[2, :, :NVP], out_den[2][:, :NVP],
                          xv, a2d(Bias[l, 2]))
        return xt_new, xv_new

    nlayers = jnp.where(ei_dep[0, 0] < -1, 3, 4)
    xt, xv = lax.fori_loop(0, nlayers, lbody, (xt0, xv0))
    return _pool_mlp(xt, W1, b1.reshape(1, 32), W2, b2.reshape(1, 32))
